# Initial kernel scaffold; baseline (speedup 1.0000x reference)
import jax
import jax.numpy as jnp
from jax.experimental import pallas as pl


def kernel(x_win, x_edge, edge_index_near, edge_index_close, edge_index_sim, ij2idx_near, ij2idx_close, ij2idx_sim, edge_edge_index, W_pre_win, W_post_win, W_pre_edge, W_post_edge, W_rel_self, W_rel_nbr, pool_q, pool_W, edge_Wl, edge_Wr, edge_b, W_out, b_out):
    raise NotImplementedError("write your pallas kernel here")


# speedup vs baseline: 2.0946x; 2.0946x over previous
import functools

import jax
import jax.numpy as jnp
from jax import lax
from jax.experimental import pallas as pl
from jax.experimental.pallas import tpu as pltpu
from jax.experimental.pallas import tpu_sc as plsc

N = 10000
NP = 10240
E = 160000
H = 512
NCH = 4
F = 128
OUT = 250
KE_PAD = 4224

NC, NS = 2, 16
PER_SUB = E // NS
NBLK = PER_SUB // 128
ROWS_PER_SUB = NP // NS

_F32 = jnp.float32


def _leaky(x):
    return jnp.where(x > 0, x, 0.2 * x)


def _pre_body(x_ref, w1_ref, w2_ref, out_ref):
    m = pl.program_id(0)
    h = _leaky(jnp.dot(x_ref[...], w1_ref[...], preferred_element_type=_F32))
    y = _leaky(jnp.dot(h, w2_ref[...], preferred_element_type=_F32))
    rows = m * 256 + lax.broadcasted_iota(jnp.int32, (256, 1), 0)
    y = jnp.where(rows < N, y, 0.0)
    for c in range(NCH):
        out_ref[c] = y[:, c * F:(c + 1) * F]


@functools.lru_cache(maxsize=None)
def _make_pre(k_dim, interpret=False):
    return pl.pallas_call(
        _pre_body,
        grid=(NP // 256,),
        in_specs=[
            pl.BlockSpec((256, k_dim), lambda m: (m, 0)),
            pl.BlockSpec((k_dim, H), lambda m: (0, 0)),
            pl.BlockSpec((H, H), lambda m: (0, 0)),
        ],
        out_specs=pl.BlockSpec((NCH, 256, F), lambda m: (0, m, 0)),
        out_shape=jax.ShapeDtypeStruct((NCH, NP, F), _F32),
        interpret=interpret,
    )


def _pool_body(xw_ref, q_ref, pw_ref, wn_ref, out_ref):
    rows = lax.broadcasted_iota(jnp.int32, (NP, 1), 0)
    mask = rows < N
    scores = jnp.zeros((NP, 1), _F32)
    for c in range(NCH):
        qc = q_ref[0, c * F:(c + 1) * F][:, None]
        scores += jnp.dot(xw_ref[c], qc, preferred_element_type=_F32)
    scores = scores * (1.0 / jnp.sqrt(jnp.float32(H)))
    sm = jnp.where(mask, scores, -1e30)
    mx = jnp.max(sm)
    e = jnp.where(mask, jnp.exp(sm - mx), 0.0)
    att = e / jnp.sum(e)
    g = jnp.zeros((1, H), _F32)
    for c in range(NCH):
        gpre = jnp.sum(att * xw_ref[c], axis=0, keepdims=True)
        g += jnp.dot(gpre, pw_ref[c * F:(c + 1) * F, :],
                     preferred_element_type=_F32)
    gws = [jnp.dot(g, wn_ref[r], preferred_element_type=_F32)
           for r in range(3)]
    out_ref[...] = jnp.concatenate(gws + [jnp.zeros((5, H), _F32)], axis=0)


@functools.lru_cache(maxsize=None)
def _make_pool(interpret=False):
    return pl.pallas_call(
        _pool_body,
        in_specs=[
            pl.BlockSpec((NCH, NP, F), lambda: (0, 0, 0)),
            pl.BlockSpec((1, H), lambda: (0, 0)),
            pl.BlockSpec((H, H), lambda: (0, 0)),
            pl.BlockSpec((3, H, H), lambda: (0, 0, 0)),
        ],
        out_specs=pl.BlockSpec((8, H), lambda: (0, 0)),
        out_shape=jax.ShapeDtypeStruct((8, H), _F32),
        interpret=interpret,
    )


def _combine_body(xw_ref, s0_ref, s1_ref, s2_ref, cnt_ref, wself_ref,
                  wnbr_ref, gw_ref, out_ref):
    m = pl.program_id(0)
    ws = wself_ref[0] + wself_ref[1] + wself_ref[2]
    acc = jnp.zeros((256, F), _F32)
    for c in range(NCH):
        acc += jnp.dot(xw_ref[c], ws[c * F:(c + 1) * F, :],
                       preferred_element_type=_F32)
    s_refs = (s0_ref, s1_ref, s2_ref)
    for r in range(3):
        cr = cnt_ref[r, :, 0:1]
        invc = 1.0 / jnp.maximum(cr, 1.0)
        dr = jnp.where(cr > 0, 1.0, 0.0)
        for c in range(NCH):
            acc += jnp.dot(s_refs[r][c] * invc,
                           wnbr_ref[r, c * F:(c + 1) * F, :],
                           preferred_element_type=_F32)
        acc += dr * gw_ref[r, :][None, :]
    y = _leaky(acc * (1.0 / 3.0))
    rows = m * 256 + lax.broadcasted_iota(jnp.int32, (256, 1), 0)
    out_ref[0] = jnp.where(rows < N, y, 0.0)


@functools.lru_cache(maxsize=None)
def _make_combine(interpret=False):
    sblock = pl.BlockSpec((NCH, 256, F), lambda m, n: (0, m, 0))
    return pl.pallas_call(
        _combine_body,
        grid=(NP // 256, NCH),
        in_specs=[
            sblock, sblock, sblock, sblock,
            pl.BlockSpec((NCH, 256, F), lambda m, n: (0, m, 0)),
            pl.BlockSpec((3, H, F), lambda m, n: (0, 0, n)),
            pl.BlockSpec((3, H, F), lambda m, n: (0, 0, n)),
            pl.BlockSpec((8, F), lambda m, n: (0, n)),
        ],
        out_specs=pl.BlockSpec((1, 256, F), lambda m, n: (n, m, 0)),
        out_shape=jax.ShapeDtypeStruct((NCH, NP, F), _F32),
        interpret=interpret,
    )


def _edge_upd_body(t_ref, cnt_ref, xe_ref, wl_ref, wr_ref, b_ref, out_ref):
    m = pl.program_id(0)
    cr = cnt_ref[3, :, 0:1]
    invc = 1.0 / jnp.maximum(cr, 1.0)
    acc = jnp.zeros((256, F), _F32)
    for c in range(NCH):
        acc += jnp.dot(t_ref[c] * invc, wl_ref[c * F:(c + 1) * F, :],
                       preferred_element_type=_F32)
        acc += jnp.dot(xe_ref[c], wr_ref[c * F:(c + 1) * F, :],
                       preferred_element_type=_F32)
    y = _leaky(acc + b_ref[...])
    rows = m * 256 + lax.broadcasted_iota(jnp.int32, (256, 1), 0)
    out_ref[0] = jnp.where(rows < N, y, 0.0)


@functools.lru_cache(maxsize=None)
def _make_edge_upd(interpret=False):
    cblock = pl.BlockSpec((NCH, 256, F), lambda m, n: (0, m, 0))
    return pl.pallas_call(
        _edge_upd_body,
        grid=(NP // 256, NCH),
        in_specs=[
            cblock,
            pl.BlockSpec((NCH, 256, F), lambda m, n: (0, m, 0)),
            cblock,
            pl.BlockSpec((H, F), lambda m, n: (0, n)),
            pl.BlockSpec((H, F), lambda m, n: (0, n)),
            pl.BlockSpec((1, F), lambda m, n: (0, n)),
        ],
        out_specs=pl.BlockSpec((1, 256, F), lambda m, n: (n, m, 0)),
        out_shape=jax.ShapeDtypeStruct((NCH, NP, F), _F32),
        interpret=interpret,
    )


def _out_body(x_ref, w_ref, b_ref, out_ref):
    acc = jnp.zeros((256, OUT), _F32)
    for c in range(NCH):
        acc += jnp.dot(x_ref[c], w_ref[c * F:(c + 1) * F, :],
                       preferred_element_type=_F32)
    out_ref[...] = acc + b_ref[...]


@functools.lru_cache(maxsize=None)
def _make_out(interpret=False):
    return pl.pallas_call(
        _out_body,
        grid=(NP // 256,),
        in_specs=[
            pl.BlockSpec((NCH, 256, F), lambda m: (0, m, 0)),
            pl.BlockSpec((H, OUT), lambda m: (0, 0)),
            pl.BlockSpec((1, OUT), lambda m: (0, 0)),
        ],
        out_specs=pl.BlockSpec((256, OUT), lambda m: (m, 0)),
        out_shape=jax.ShapeDtypeStruct((N, OUT), _F32),
        interpret=interpret,
    )


@functools.lru_cache(maxsize=None)
def _mesh():
    return plsc.VectorSubcoreMesh(core_axis_name="c", subcore_axis_name="s",
                                  num_cores=NC, num_subcores=NS)


def _fill_rows(ref, nrows, value):
    width = ref.shape[1]

    def body(i, _):
        for j in range(width // 16):
            ref[i, pl.ds(j * 16, 16)] = jnp.full((16,), value, _F32)
        return 0

    lax.fori_loop(0, nrows, body, 0)


def _zero_acc(zrow, acc, s, width_rows):
    nz = zrow.shape[0]

    def body(i, _):
        pltpu.sync_copy(zrow, acc.at[pl.ds(s * ROWS_PER_SUB + i * nz, nz), :])
        return 0

    lax.fori_loop(0, ROWS_PER_SUB // nz, body, 0)


def _segsum_body(two_tables, *refs):
    if two_tables:
        (tab_a, idx_a, tab_b, idx_b, dst, out,
         ia, ib, dv, ia16, ib16, dv16, ra, rb, ra16, rb16, zrow, acc) = refs
    else:
        (tab_a, idx_a, dst, out,
         ia, dv, ia16, dv16, ra, ra16, zrow, acc) = refs
    core = lax.axis_index("c")
    s = lax.axis_index("s")
    _fill_rows(zrow, zrow.shape[0], 0.0)

    for k in range(2):
        chunk = core * 2 + k
        off = chunk * NP
        _zero_acc(zrow, acc, s, ROWS_PER_SUB)
        plsc.subcore_barrier()

        def blk(b, _):
            base = s * PER_SUB + b * 128
            pltpu.sync_copy(idx_a.at[pl.ds(base, 128)], ia)
            pltpu.sync_copy(dst.at[pl.ds(base, 128)], dv)
            for j in range(8):
                sl = pl.ds(j * 16, 16)
                ia[sl] = ia[sl] + off
            pltpu.sync_copy(tab_a.at[ia], ra)
            if two_tables:
                pltpu.sync_copy(idx_b.at[pl.ds(base, 128)], ib)
                for j in range(8):
                    sl = pl.ds(j * 16, 16)
                    ib[sl] = ib[sl] + off
                pltpu.sync_copy(tab_b.at[ib], rb)

                def addrow(i, _):
                    for j in range(8):
                        sl2 = pl.ds(j * 16, 16)
                        ra[i, sl2] = ra[i, sl2] + rb[i, sl2]
                    return 0

                lax.fori_loop(0, 128, addrow, 0)
            pltpu.sync_copy(ra, acc.at[dv], add=True)
            return 0

        lax.fori_loop(0, NBLK, blk, 0)

        base = s * PER_SUB + NBLK * 128
        pltpu.sync_copy(idx_a.at[pl.ds(base, 16)], ia16)
        pltpu.sync_copy(dst.at[pl.ds(base, 16)], dv16)
        ia16[...] = ia16[...] + off
        pltpu.sync_copy(tab_a.at[ia16], ra16)
        if two_tables:
            pltpu.sync_copy(idx_b.at[pl.ds(base, 16)], ib16)
            ib16[...] = ib16[...] + off
            pltpu.sync_copy(tab_b.at[ib16], rb16)

            def addrow16(i, _):
                for j in range(8):
                    sl2 = pl.ds(j * 16, 16)
                    ra16[i, sl2] = ra16[i, sl2] + rb16[i, sl2]
                return 0

            lax.fori_loop(0, 16, addrow16, 0)
        pltpu.sync_copy(ra16, acc.at[dv16], add=True)

        plsc.subcore_barrier()
        pltpu.sync_copy(acc.at[pl.ds(s * ROWS_PER_SUB, ROWS_PER_SUB), :],
                        out.at[chunk, pl.ds(s * ROWS_PER_SUB, ROWS_PER_SUB), :])
        plsc.subcore_barrier()


@functools.lru_cache(maxsize=None)
def _make_segsum(two_tables, interpret=False):
    scratch = [
        pltpu.VMEM((128,), jnp.int32),
    ]
    if two_tables:
        scratch.append(pltpu.VMEM((128,), jnp.int32))
    scratch.append(pltpu.VMEM((128,), jnp.int32))
    scratch.append(pltpu.VMEM((16,), jnp.int32))
    if two_tables:
        scratch.append(pltpu.VMEM((16,), jnp.int32))
    scratch.append(pltpu.VMEM((16,), jnp.int32))
    scratch.append(pltpu.VMEM((128, F), _F32))
    if two_tables:
        scratch.append(pltpu.VMEM((128, F), _F32))
    scratch.append(pltpu.VMEM((16, F), _F32))
    if two_tables:
        scratch.append(pltpu.VMEM((16, F), _F32))
    scratch.append(pltpu.VMEM((64, F), _F32))
    scratch.append(pltpu.VMEM_SHARED((NP, F), _F32))

    if two_tables:
        order = scratch
    else:
        order = scratch
    return pl.kernel(
        functools.partial(_segsum_body, two_tables),
        out_type=jax.ShapeDtypeStruct((NCH, NP, F), _F32),
        mesh=_mesh(),
        scratch_types=order,
        interpret=interpret,
    )


def _count_body(d0, d1, d2, d3, out, dv, dv16, ones, ones16, zrow, acc):
    core = lax.axis_index("c")
    s = lax.axis_index("s")
    _fill_rows(zrow, zrow.shape[0], 0.0)
    _fill_rows(ones, 128, 1.0)
    _fill_rows(ones16, 16, 1.0)
    dsts = (d0, d1, d2, d3)

    for k in range(2):
        job = core * 2 + k
        _zero_acc(zrow, acc, s, ROWS_PER_SUB)
        plsc.subcore_barrier()
        for jj in range(4):
            @pl.when(job == jj)
            def _scan(dref=dsts[jj]):
                def blk(b, _):
                    base = s * PER_SUB + b * 128
                    pltpu.sync_copy(dref.at[pl.ds(base, 128)], dv)
                    pltpu.sync_copy(ones, acc.at[dv], add=True)
                    return 0
                lax.fori_loop(0, NBLK, blk, 0)
                base = s * PER_SUB + NBLK * 128
                pltpu.sync_copy(dref.at[pl.ds(base, 16)], dv16)
                pltpu.sync_copy(ones16, acc.at[dv16], add=True)
        plsc.subcore_barrier()
        pltpu.sync_copy(acc.at[pl.ds(s * ROWS_PER_SUB, ROWS_PER_SUB), :],
                        out.at[job, pl.ds(s * ROWS_PER_SUB, ROWS_PER_SUB), :])
        plsc.subcore_barrier()


@functools.lru_cache(maxsize=None)
def _make_count(interpret=False):
    return pl.kernel(
        _count_body,
        out_type=jax.ShapeDtypeStruct((4, NP, F), _F32),
        mesh=_mesh(),
        scratch_types=[
            pltpu.VMEM((128,), jnp.int32),
            pltpu.VMEM((16,), jnp.int32),
            pltpu.VMEM((128, F), _F32),
            pltpu.VMEM((16, F), _F32),
            pltpu.VMEM((64, F), _F32),
            pltpu.VMEM_SHARED((NP, F), _F32),
        ],
        interpret=interpret,
    )


def kernel(x_win, x_edge, edge_index_near, edge_index_close, edge_index_sim,
           ij2idx_near, ij2idx_close, ij2idx_sim, edge_edge_index,
           W_pre_win, W_post_win, W_pre_edge, W_post_edge,
           W_rel_self, W_rel_nbr, pool_q, pool_W,
           edge_Wl, edge_Wr, edge_b, W_out, b_out):
    L = W_rel_self.shape[0]
    i32 = jnp.int32
    rels = [
        (edge_index_near[0].astype(i32), edge_index_near[1].astype(i32),
         ij2idx_near.astype(i32)),
        (edge_index_close[0].astype(i32), edge_index_close[1].astype(i32),
         ij2idx_close.astype(i32)),
        (edge_index_sim[0].astype(i32), edge_index_sim[1].astype(i32),
         ij2idx_sim.astype(i32)),
    ]
    es = edge_edge_index[0].astype(i32)
    ed = edge_edge_index[1].astype(i32)

    xe_pad = jnp.pad(x_edge, ((0, 0), (0, KE_PAD - x_edge.shape[1])))
    w1e_pad = jnp.pad(W_pre_edge, ((0, KE_PAD - W_pre_edge.shape[0]), (0, 0)))

    xw = _make_pre(H)(x_win, W_pre_win, W_post_win)
    xe = _make_pre(KE_PAD)(xe_pad, w1e_pad, W_post_edge)

    cnt = _make_count()(rels[0][1], rels[1][1], rels[2][1], ed)

    pool = _make_pool()
    seg2 = _make_segsum(True)
    seg1 = _make_segsum(False)
    comb = _make_combine()
    eupd = _make_edge_upd()

    for l in range(L):
        gw = pool(xw, pool_q[l][None, :], pool_W[l], W_rel_nbr[l])
        xw_flat = xw.reshape(NCH * NP, F)
        xe_flat = xe.reshape(NCH * NP, F)
        s_aggr = [seg2(xw_flat, src, xe_flat, ij, dst)
                  for (src, dst, ij) in rels]
        t_aggr = seg1(xe_flat, es, ed)
        xw_new = comb(xw, s_aggr[0], s_aggr[1], s_aggr[2], cnt,
                      W_rel_self[l], W_rel_nbr[l], gw)
        xe = eupd(t_aggr, cnt, xe, edge_Wl[l], edge_Wr[l],
                  edge_b[l][None, :])
        xw = xw_new

    return _make_out()(xw, W_out, b_out[None, :])


# speedup vs baseline: 3.2708x; 1.5615x over previous
import functools

import jax
import jax.numpy as jnp
from jax import lax
from jax.experimental import pallas as pl
from jax.experimental.pallas import tpu as pltpu
from jax.experimental.pallas import tpu_sc as plsc

N = 10000
NP = 10240
E = 160000
H = 512
NCH = 4
F = 128
OUT = 250
KE_PAD = 4224

NC, NS = 2, 16
PER_SUB = E // NS
BLK = 64
NBLK_CNT = PER_SUB // 128
NBLK = PER_SUB // BLK
ROWS_PER_SUB = NP // NS

_F32 = jnp.float32


def _leaky(x):
    return jnp.where(x > 0, x, 0.2 * x)


def _pre_body(x_ref, w1_ref, w2_ref, out_ref):
    m = pl.program_id(0)
    h = _leaky(jnp.dot(x_ref[...], w1_ref[...], preferred_element_type=_F32))
    y = _leaky(jnp.dot(h, w2_ref[...], preferred_element_type=_F32))
    rows = m * 256 + lax.broadcasted_iota(jnp.int32, (256, 1), 0)
    y = jnp.where(rows < N, y, 0.0)
    for c in range(NCH):
        out_ref[c] = y[:, c * F:(c + 1) * F]


@functools.lru_cache(maxsize=None)
def _make_pre(k_dim, interpret=False):
    return pl.pallas_call(
        _pre_body,
        grid=(NP // 256,),
        in_specs=[
            pl.BlockSpec((256, k_dim), lambda m: (m, 0)),
            pl.BlockSpec((k_dim, H), lambda m: (0, 0)),
            pl.BlockSpec((H, H), lambda m: (0, 0)),
        ],
        out_specs=pl.BlockSpec((NCH, 256, F), lambda m: (0, m, 0)),
        out_shape=jax.ShapeDtypeStruct((NCH, NP, F), _F32),
        interpret=interpret,
    )


def _pool_body(xw_ref, q_ref, pw_ref, wn_ref, out_ref):
    rows = lax.broadcasted_iota(jnp.int32, (NP, 1), 0)
    mask = rows < N
    scores = jnp.zeros((NP, 1), _F32)
    for c in range(NCH):
        qc = q_ref[0, c * F:(c + 1) * F][:, None]
        scores += jnp.dot(xw_ref[c], qc, preferred_element_type=_F32)
    scores = scores * (1.0 / jnp.sqrt(jnp.float32(H)))
    sm = jnp.where(mask, scores, -1e30)
    mx = jnp.max(sm)
    e = jnp.where(mask, jnp.exp(sm - mx), 0.0)
    att = e / jnp.sum(e)
    g = jnp.zeros((1, H), _F32)
    for c in range(NCH):
        gpre = jnp.sum(att * xw_ref[c], axis=0, keepdims=True)
        g += jnp.dot(gpre, pw_ref[c * F:(c + 1) * F, :],
                     preferred_element_type=_F32)
    gws = [jnp.dot(g, wn_ref[r], preferred_element_type=_F32)
           for r in range(3)]
    out_ref[...] = jnp.concatenate(gws + [jnp.zeros((5, H), _F32)], axis=0)


@functools.lru_cache(maxsize=None)
def _make_pool(interpret=False):
    return pl.pallas_call(
        _pool_body,
        in_specs=[
            pl.BlockSpec((NCH, NP, F), lambda: (0, 0, 0)),
            pl.BlockSpec((1, H), lambda: (0, 0)),
            pl.BlockSpec((H, H), lambda: (0, 0)),
            pl.BlockSpec((3, H, H), lambda: (0, 0, 0)),
        ],
        out_specs=pl.BlockSpec((8, H), lambda: (0, 0)),
        out_shape=jax.ShapeDtypeStruct((8, H), _F32),
        interpret=interpret,
    )


def _combine_body(xw_ref, s0_ref, s1_ref, s2_ref, cnt_ref, wself_ref,
                  wnbr_ref, gw_ref, out_ref):
    m = pl.program_id(0)
    ws = wself_ref[0] + wself_ref[1] + wself_ref[2]
    acc = jnp.zeros((256, F), _F32)
    for c in range(NCH):
        acc += jnp.dot(xw_ref[c], ws[c * F:(c + 1) * F, :],
                       preferred_element_type=_F32)
    s_refs = (s0_ref, s1_ref, s2_ref)
    for r in range(3):
        cr = cnt_ref[r, :, 0:1]
        invc = 1.0 / jnp.maximum(cr, 1.0)
        dr = jnp.where(cr > 0, 1.0, 0.0)
        for c in range(NCH):
            acc += jnp.dot(s_refs[r][c] * invc,
                           wnbr_ref[r, c * F:(c + 1) * F, :],
                           preferred_element_type=_F32)
        acc += dr * gw_ref[r, :][None, :]
    y = _leaky(acc * (1.0 / 3.0))
    rows = m * 256 + lax.broadcasted_iota(jnp.int32, (256, 1), 0)
    out_ref[0] = jnp.where(rows < N, y, 0.0)


@functools.lru_cache(maxsize=None)
def _make_combine(interpret=False):
    sblock = pl.BlockSpec((NCH, 256, F), lambda m, n: (0, m, 0))
    return pl.pallas_call(
        _combine_body,
        grid=(NP // 256, NCH),
        in_specs=[
            sblock, sblock, sblock, sblock,
            pl.BlockSpec((NCH, 256, F), lambda m, n: (0, m, 0)),
            pl.BlockSpec((3, H, F), lambda m, n: (0, 0, n)),
            pl.BlockSpec((3, H, F), lambda m, n: (0, 0, n)),
            pl.BlockSpec((8, F), lambda m, n: (0, n)),
        ],
        out_specs=pl.BlockSpec((1, 256, F), lambda m, n: (n, m, 0)),
        out_shape=jax.ShapeDtypeStruct((NCH, NP, F), _F32),
        interpret=interpret,
    )


def _edge_upd_body(t_ref, cnt_ref, xe_ref, wl_ref, wr_ref, b_ref, out_ref):
    m = pl.program_id(0)
    cr = cnt_ref[3, :, 0:1]
    invc = 1.0 / jnp.maximum(cr, 1.0)
    acc = jnp.zeros((256, F), _F32)
    for c in range(NCH):
        acc += jnp.dot(t_ref[c] * invc, wl_ref[c * F:(c + 1) * F, :],
                       preferred_element_type=_F32)
        acc += jnp.dot(xe_ref[c], wr_ref[c * F:(c + 1) * F, :],
                       preferred_element_type=_F32)
    y = _leaky(acc + b_ref[...])
    rows = m * 256 + lax.broadcasted_iota(jnp.int32, (256, 1), 0)
    out_ref[0] = jnp.where(rows < N, y, 0.0)


@functools.lru_cache(maxsize=None)
def _make_edge_upd(interpret=False):
    cblock = pl.BlockSpec((NCH, 256, F), lambda m, n: (0, m, 0))
    return pl.pallas_call(
        _edge_upd_body,
        grid=(NP // 256, NCH),
        in_specs=[
            cblock,
            pl.BlockSpec((NCH, 256, F), lambda m, n: (0, m, 0)),
            cblock,
            pl.BlockSpec((H, F), lambda m, n: (0, n)),
            pl.BlockSpec((H, F), lambda m, n: (0, n)),
            pl.BlockSpec((1, F), lambda m, n: (0, n)),
        ],
        out_specs=pl.BlockSpec((1, 256, F), lambda m, n: (n, m, 0)),
        out_shape=jax.ShapeDtypeStruct((NCH, NP, F), _F32),
        interpret=interpret,
    )


def _out_body(x_ref, w_ref, b_ref, out_ref):
    acc = jnp.zeros((256, OUT), _F32)
    for c in range(NCH):
        acc += jnp.dot(x_ref[c], w_ref[c * F:(c + 1) * F, :],
                       preferred_element_type=_F32)
    out_ref[...] = acc + b_ref[...]


@functools.lru_cache(maxsize=None)
def _make_out(interpret=False):
    return pl.pallas_call(
        _out_body,
        grid=(NP // 256,),
        in_specs=[
            pl.BlockSpec((NCH, 256, F), lambda m: (0, m, 0)),
            pl.BlockSpec((H, OUT), lambda m: (0, 0)),
            pl.BlockSpec((1, OUT), lambda m: (0, 0)),
        ],
        out_specs=pl.BlockSpec((256, OUT), lambda m: (m, 0)),
        out_shape=jax.ShapeDtypeStruct((N, OUT), _F32),
        interpret=interpret,
    )


@functools.lru_cache(maxsize=None)
def _mesh():
    return plsc.VectorSubcoreMesh(core_axis_name="c", subcore_axis_name="s",
                                  num_cores=NC, num_subcores=NS)


def _fill_rows(ref, nrows, value):
    width = ref.shape[1]

    def body(i, _):
        for j in range(width // 16):
            ref[i, pl.ds(j * 16, 16)] = jnp.full((16,), value, _F32)
        return 0

    lax.fori_loop(0, nrows, body, 0)


def _zero_acc(zrow, acc, s, width_rows):
    nz = zrow.shape[0]

    def body(i, _):
        pltpu.sync_copy(zrow, acc.at[pl.ds(s * ROWS_PER_SUB + i * nz, nz), :])
        return 0

    lax.fori_loop(0, ROWS_PER_SUB // nz, body, 0)


NSLOT = 2
NGRP = NBLK // NSLOT


def _segsum_body(two_tables, *refs):
    n_in = 5 if two_tables else 3
    tabs = refs[:n_in]
    out = refs[n_in]
    rest = list(refs[n_in + 1:])
    if two_tables:
        tab_a, idx_a, tab_b, idx_b, dst = tabs
    else:
        tab_a, idx_a, dst = tabs

    def take(n):
        nonlocal rest
        got, rest = rest[:n], rest[n:]
        return got

    ia = take(NSLOT)
    ib = take(NSLOT) if two_tables else None
    dv = take(NSLOT)
    ia16, = take(1)
    if two_tables:
        ib16, = take(1)
    dv16, = take(1)
    ra = take(NSLOT)
    rb = take(NSLOT) if two_tables else None
    ra16, = take(1)
    if two_tables:
        rb16, = take(1)
    zrow, = take(1)
    acc, = take(1)
    sidx = take(NSLOT)
    sga = take(NSLOT)
    sgb = take(NSLOT) if two_tables else None
    ssc = take(NSLOT)
    core = lax.axis_index("c")
    sid = lax.axis_index("s")
    _fill_rows(zrow, zrow.shape[0], 0.0)

    for k in range(2):
        chunk = core * 2 + k
        off = chunk * NP
        _zero_acc(zrow, acc, sid, ROWS_PER_SUB)
        plsc.subcore_barrier()

        def issue_idx(bb, t):
            base = sid * PER_SUB + bb * BLK
            pltpu.make_async_copy(idx_a.at[pl.ds(base, BLK)], ia[t],
                                  sidx[t]).start()
            if two_tables:
                pltpu.make_async_copy(idx_b.at[pl.ds(base, BLK)], ib[t],
                                      sidx[t]).start()
            pltpu.make_async_copy(dst.at[pl.ds(base, BLK)], dv[t],
                                  sidx[t]).start()

        def wait_idx_offset(bb, t):
            base = sid * PER_SUB + bb * BLK
            pltpu.make_async_copy(idx_a.at[pl.ds(base, BLK)], ia[t],
                                  sidx[t]).wait()
            if two_tables:
                pltpu.make_async_copy(idx_b.at[pl.ds(base, BLK)], ib[t],
                                      sidx[t]).wait()
            pltpu.make_async_copy(dst.at[pl.ds(base, BLK)], dv[t],
                                  sidx[t]).wait()
            for j in range(BLK // 16):
                sl = pl.ds(j * 16, 16)
                ia[t][sl] = ia[t][sl] + off
                if two_tables:
                    ib[t][sl] = ib[t][sl] + off

        def issue_gathers(t):
            pltpu.make_async_copy(tab_a.at[ia[t]], ra[t], sga[t]).start()
            if two_tables:
                pltpu.make_async_copy(tab_b.at[ib[t]], rb[t], sgb[t]).start()

        def process(t):
            pltpu.make_async_copy(tab_a.at[ia[t]], ra[t], sga[t]).wait()
            if two_tables:
                pltpu.make_async_copy(tab_b.at[ib[t]], rb[t], sgb[t]).wait()

                def addrow(i, _):
                    for j in range(8):
                        sl2 = pl.ds(j * 16, 16)
                        ra[t][i, sl2] = ra[t][i, sl2] + rb[t][i, sl2]
                    return 0

                lax.fori_loop(0, BLK, addrow, 0)
            pltpu.make_async_copy(ra[t], acc.at[dv[t]],
                                  ssc[t]).start(add=True)

        def wait_sc(t):
            pltpu.make_async_copy(ra[t], acc.at[dv[t]], ssc[t]).wait()

        for t in range(NSLOT):
            issue_idx(t, t)
            wait_idx_offset(t, t)
            issue_gathers(t)

        def grp(i, _):
            for t in range(NSLOT):
                process(t)
            nxt = (i + 1) * NSLOT
            nxt = jnp.where(nxt >= NBLK, 0, nxt)
            for t in range(NSLOT):
                wait_sc(t)
                issue_idx(nxt + t, t)
                wait_idx_offset(nxt + t, t)
                issue_gathers(t)
            return 0

        lax.fori_loop(0, NGRP, grp, 0)

        for t in range(NSLOT):
            pltpu.make_async_copy(tab_a.at[ia[t]], ra[t], sga[t]).wait()
            if two_tables:
                pltpu.make_async_copy(tab_b.at[ib[t]], rb[t], sgb[t]).wait()

        base = sid * PER_SUB + NBLK * BLK
        pltpu.sync_copy(idx_a.at[pl.ds(base, 16)], ia16)
        pltpu.sync_copy(dst.at[pl.ds(base, 16)], dv16)
        ia16[...] = ia16[...] + off
        pltpu.sync_copy(tab_a.at[ia16], ra16)
        if two_tables:
            pltpu.sync_copy(idx_b.at[pl.ds(base, 16)], ib16)
            ib16[...] = ib16[...] + off
            pltpu.sync_copy(tab_b.at[ib16], rb16)

            def addrow16(i, _):
                for j in range(8):
                    sl2 = pl.ds(j * 16, 16)
                    ra16[i, sl2] = ra16[i, sl2] + rb16[i, sl2]
                return 0

            lax.fori_loop(0, 16, addrow16, 0)
        pltpu.sync_copy(ra16, acc.at[dv16], add=True)

        plsc.subcore_barrier()
        pltpu.sync_copy(acc.at[pl.ds(sid * ROWS_PER_SUB, ROWS_PER_SUB), :],
                        out.at[chunk,
                               pl.ds(sid * ROWS_PER_SUB, ROWS_PER_SUB), :])
        plsc.subcore_barrier()


def _make_segsum(two_tables, interpret=False):
    i32v = pltpu.VMEM((BLK,), jnp.int32)
    i16v = pltpu.VMEM((16,), jnp.int32)
    rowv = pltpu.VMEM((BLK, F), _F32)
    row16 = pltpu.VMEM((16, F), _F32)
    sem = pltpu.SemaphoreType.DMA
    scratch = [i32v] * NSLOT
    if two_tables:
        scratch += [i32v] * NSLOT
    scratch += [i32v] * NSLOT
    scratch += [i16v] * (3 if two_tables else 2)
    scratch += [rowv] * NSLOT
    if two_tables:
        scratch += [rowv] * NSLOT
    scratch += [row16] * (2 if two_tables else 1)
    scratch += [pltpu.VMEM((16, F), _F32)]
    scratch += [pltpu.VMEM_SHARED((NP, F), _F32)]
    scratch += [sem] * (NSLOT * (4 if two_tables else 3))
    return pl.kernel(
        functools.partial(_segsum_body, two_tables),
        out_type=jax.ShapeDtypeStruct((NCH, NP, F), _F32),
        mesh=_mesh(),
        scratch_types=scratch,
        interpret=interpret,
    )


def _count_body(d0, d1, d2, d3, out, dv, dv16, ones, ones16, zrow, acc):
    core = lax.axis_index("c")
    s = lax.axis_index("s")
    _fill_rows(zrow, zrow.shape[0], 0.0)
    _fill_rows(ones, 128, 1.0)
    _fill_rows(ones16, 16, 1.0)
    dsts = (d0, d1, d2, d3)

    for k in range(2):
        job = core * 2 + k
        _zero_acc(zrow, acc, s, ROWS_PER_SUB)
        plsc.subcore_barrier()
        for jj in range(4):
            @pl.when(job == jj)
            def _scan(dref=dsts[jj]):
                def blk(b, _):
                    base = s * PER_SUB + b * 128
                    pltpu.sync_copy(dref.at[pl.ds(base, 128)], dv)
                    pltpu.sync_copy(ones, acc.at[dv], add=True)
                    return 0
                lax.fori_loop(0, NBLK_CNT, blk, 0)
                base = s * PER_SUB + NBLK_CNT * 128
                pltpu.sync_copy(dref.at[pl.ds(base, 16)], dv16)
                pltpu.sync_copy(ones16, acc.at[dv16], add=True)
        plsc.subcore_barrier()
        pltpu.sync_copy(acc.at[pl.ds(s * ROWS_PER_SUB, ROWS_PER_SUB), :],
                        out.at[job, pl.ds(s * ROWS_PER_SUB, ROWS_PER_SUB), :])
        plsc.subcore_barrier()


@functools.lru_cache(maxsize=None)
def _make_count(interpret=False):
    return pl.kernel(
        _count_body,
        out_type=jax.ShapeDtypeStruct((4, NP, F), _F32),
        mesh=_mesh(),
        scratch_types=[
            pltpu.VMEM((128,), jnp.int32),
            pltpu.VMEM((16,), jnp.int32),
            pltpu.VMEM((128, F), _F32),
            pltpu.VMEM((16, F), _F32),
            pltpu.VMEM((64, F), _F32),
            pltpu.VMEM_SHARED((NP, F), _F32),
        ],
        interpret=interpret,
    )


def kernel(x_win, x_edge, edge_index_near, edge_index_close, edge_index_sim,
           ij2idx_near, ij2idx_close, ij2idx_sim, edge_edge_index,
           W_pre_win, W_post_win, W_pre_edge, W_post_edge,
           W_rel_self, W_rel_nbr, pool_q, pool_W,
           edge_Wl, edge_Wr, edge_b, W_out, b_out):
    L = W_rel_self.shape[0]
    i32 = jnp.int32
    rels = [
        (edge_index_near[0].astype(i32), edge_index_near[1].astype(i32),
         ij2idx_near.astype(i32)),
        (edge_index_close[0].astype(i32), edge_index_close[1].astype(i32),
         ij2idx_close.astype(i32)),
        (edge_index_sim[0].astype(i32), edge_index_sim[1].astype(i32),
         ij2idx_sim.astype(i32)),
    ]
    es = edge_edge_index[0].astype(i32)
    ed = edge_edge_index[1].astype(i32)

    xe_pad = jnp.pad(x_edge, ((0, 0), (0, KE_PAD - x_edge.shape[1])))
    w1e_pad = jnp.pad(W_pre_edge, ((0, KE_PAD - W_pre_edge.shape[0]), (0, 0)))

    xw = _make_pre(H)(x_win, W_pre_win, W_post_win)
    xe = _make_pre(KE_PAD)(xe_pad, w1e_pad, W_post_edge)

    cnt = _make_count()(rels[0][1], rels[1][1], rels[2][1], ed)

    pool = _make_pool()
    seg2 = _make_segsum(True)
    seg1 = _make_segsum(False)
    comb = _make_combine()
    eupd = _make_edge_upd()

    for l in range(L):
        gw = pool(xw, pool_q[l][None, :], pool_W[l], W_rel_nbr[l])
        xw_flat = xw.reshape(NCH * NP, F)
        xe_flat = xe.reshape(NCH * NP, F)
        s_aggr = [seg2(xw_flat, src, xe_flat, ij, dst)
                  for (src, dst, ij) in rels]
        t_aggr = seg1(xe_flat, es, ed)
        xw_new = comb(xw, s_aggr[0], s_aggr[1], s_aggr[2], cnt,
                      W_rel_self[l], W_rel_nbr[l], gw)
        xe = eupd(t_aggr, cnt, xe, edge_Wl[l], edge_Wr[l],
                  edge_b[l][None, :])
        xw = xw_new

    return _make_out()(xw, W_out, b_out[None, :])


# speedup vs baseline: 3.5822x; 1.0952x over previous
import functools

import jax
import jax.numpy as jnp
from jax import lax
from jax.experimental import pallas as pl
from jax.experimental.pallas import tpu as pltpu
from jax.experimental.pallas import tpu_sc as plsc

N = 10000
NP = 10240
E = 160000
H = 512
NCH = 4
F = 128
OUT = 250
KE_PAD = 4224

NC, NS = 2, 16
PER_SUB = E // NS
BLK = 64
NBLK_CNT = PER_SUB // 128
NBLK = PER_SUB // BLK
ROWS_PER_SUB = NP // NS

_F32 = jnp.float32


def _leaky(x):
    return jnp.where(x > 0, x, 0.2 * x)


def _pre_body(x_ref, w1_ref, w2_ref, out_ref):
    m = pl.program_id(0)
    h = _leaky(jnp.dot(x_ref[...], w1_ref[...], preferred_element_type=_F32))
    y = _leaky(jnp.dot(h, w2_ref[...], preferred_element_type=_F32))
    rows = m * 256 + lax.broadcasted_iota(jnp.int32, (256, 1), 0)
    y = jnp.where(rows < N, y, 0.0)
    for c in range(NCH):
        out_ref[c] = y[:, c * F:(c + 1) * F]


@functools.lru_cache(maxsize=None)
def _make_pre(k_dim, interpret=False):
    return pl.pallas_call(
        _pre_body,
        grid=(NP // 256,),
        in_specs=[
            pl.BlockSpec((256, k_dim), lambda m: (m, 0)),
            pl.BlockSpec((k_dim, H), lambda m: (0, 0)),
            pl.BlockSpec((H, H), lambda m: (0, 0)),
        ],
        out_specs=pl.BlockSpec((NCH, 256, F), lambda m: (0, m, 0)),
        out_shape=jax.ShapeDtypeStruct((NCH, NP, F), _F32),
        interpret=interpret,
    )


def _pool_body(xw_ref, q_ref, pw_ref, wn_ref, out_ref):
    rows = lax.broadcasted_iota(jnp.int32, (NP, 1), 0)
    mask = rows < N
    scores = jnp.zeros((NP, 1), _F32)
    for c in range(NCH):
        qc = q_ref[0, c * F:(c + 1) * F][:, None]
        scores += jnp.dot(xw_ref[c], qc, preferred_element_type=_F32)
    scores = scores * (1.0 / jnp.sqrt(jnp.float32(H)))
    sm = jnp.where(mask, scores, -1e30)
    mx = jnp.max(sm)
    e = jnp.where(mask, jnp.exp(sm - mx), 0.0)
    att = e / jnp.sum(e)
    g = jnp.zeros((1, H), _F32)
    for c in range(NCH):
        gpre = jnp.sum(att * xw_ref[c], axis=0, keepdims=True)
        g += jnp.dot(gpre, pw_ref[c * F:(c + 1) * F, :],
                     preferred_element_type=_F32)
    gws = [jnp.dot(g, wn_ref[r], preferred_element_type=_F32)
           for r in range(3)]
    out_ref[...] = jnp.concatenate(gws + [jnp.zeros((5, H), _F32)], axis=0)


@functools.lru_cache(maxsize=None)
def _make_pool(interpret=False):
    return pl.pallas_call(
        _pool_body,
        in_specs=[
            pl.BlockSpec((NCH, NP, F), lambda: (0, 0, 0)),
            pl.BlockSpec((1, H), lambda: (0, 0)),
            pl.BlockSpec((H, H), lambda: (0, 0)),
            pl.BlockSpec((3, H, H), lambda: (0, 0, 0)),
        ],
        out_specs=pl.BlockSpec((8, H), lambda: (0, 0)),
        out_shape=jax.ShapeDtypeStruct((8, H), _F32),
        interpret=interpret,
    )


def _combine_body(xw_ref, s0_ref, s1_ref, s2_ref, cnt_ref, wself_ref,
                  wnbr_ref, gw_ref, out_ref):
    m = pl.program_id(0)
    ws = wself_ref[0] + wself_ref[1] + wself_ref[2]
    acc = jnp.zeros((256, F), _F32)
    for c in range(NCH):
        acc += jnp.dot(xw_ref[c], ws[c * F:(c + 1) * F, :],
                       preferred_element_type=_F32)
    s_refs = (s0_ref, s1_ref, s2_ref)
    for r in range(3):
        cr = cnt_ref[r, :, 0:1]
        invc = 1.0 / jnp.maximum(cr, 1.0)
        dr = jnp.where(cr > 0, 1.0, 0.0)
        for c in range(NCH):
            acc += jnp.dot(s_refs[r][c] * invc,
                           wnbr_ref[r, c * F:(c + 1) * F, :],
                           preferred_element_type=_F32)
        acc += dr * gw_ref[r, :][None, :]
    y = _leaky(acc * (1.0 / 3.0))
    rows = m * 256 + lax.broadcasted_iota(jnp.int32, (256, 1), 0)
    out_ref[0] = jnp.where(rows < N, y, 0.0)


@functools.lru_cache(maxsize=None)
def _make_combine(interpret=False):
    sblock = pl.BlockSpec((NCH, 256, F), lambda m, n: (0, m, 0))
    return pl.pallas_call(
        _combine_body,
        grid=(NP // 256, NCH),
        in_specs=[
            sblock, sblock, sblock, sblock,
            pl.BlockSpec((NCH, 256, F), lambda m, n: (0, m, 0)),
            pl.BlockSpec((3, H, F), lambda m, n: (0, 0, n)),
            pl.BlockSpec((3, H, F), lambda m, n: (0, 0, n)),
            pl.BlockSpec((8, F), lambda m, n: (0, n)),
        ],
        out_specs=pl.BlockSpec((1, 256, F), lambda m, n: (n, m, 0)),
        out_shape=jax.ShapeDtypeStruct((NCH, NP, F), _F32),
        interpret=interpret,
    )


def _edge_upd_body(t_ref, cnt_ref, xe_ref, wl_ref, wr_ref, b_ref, out_ref):
    m = pl.program_id(0)
    cr = cnt_ref[3, :, 0:1]
    invc = 1.0 / jnp.maximum(cr, 1.0)
    acc = jnp.zeros((256, F), _F32)
    for c in range(NCH):
        acc += jnp.dot(t_ref[c] * invc, wl_ref[c * F:(c + 1) * F, :],
                       preferred_element_type=_F32)
        acc += jnp.dot(xe_ref[c], wr_ref[c * F:(c + 1) * F, :],
                       preferred_element_type=_F32)
    y = _leaky(acc + b_ref[...])
    rows = m * 256 + lax.broadcasted_iota(jnp.int32, (256, 1), 0)
    out_ref[0] = jnp.where(rows < N, y, 0.0)


@functools.lru_cache(maxsize=None)
def _make_edge_upd(interpret=False):
    cblock = pl.BlockSpec((NCH, 256, F), lambda m, n: (0, m, 0))
    return pl.pallas_call(
        _edge_upd_body,
        grid=(NP // 256, NCH),
        in_specs=[
            cblock,
            pl.BlockSpec((NCH, 256, F), lambda m, n: (0, m, 0)),
            cblock,
            pl.BlockSpec((H, F), lambda m, n: (0, n)),
            pl.BlockSpec((H, F), lambda m, n: (0, n)),
            pl.BlockSpec((1, F), lambda m, n: (0, n)),
        ],
        out_specs=pl.BlockSpec((1, 256, F), lambda m, n: (n, m, 0)),
        out_shape=jax.ShapeDtypeStruct((NCH, NP, F), _F32),
        interpret=interpret,
    )


def _out_body(x_ref, w_ref, b_ref, out_ref):
    acc = jnp.zeros((256, OUT), _F32)
    for c in range(NCH):
        acc += jnp.dot(x_ref[c], w_ref[c * F:(c + 1) * F, :],
                       preferred_element_type=_F32)
    out_ref[...] = acc + b_ref[...]


@functools.lru_cache(maxsize=None)
def _make_out(interpret=False):
    return pl.pallas_call(
        _out_body,
        grid=(NP // 256,),
        in_specs=[
            pl.BlockSpec((NCH, 256, F), lambda m: (0, m, 0)),
            pl.BlockSpec((H, OUT), lambda m: (0, 0)),
            pl.BlockSpec((1, OUT), lambda m: (0, 0)),
        ],
        out_specs=pl.BlockSpec((256, OUT), lambda m: (m, 0)),
        out_shape=jax.ShapeDtypeStruct((N, OUT), _F32),
        interpret=interpret,
    )


@functools.lru_cache(maxsize=None)
def _mesh():
    return plsc.VectorSubcoreMesh(core_axis_name="c", subcore_axis_name="s",
                                  num_cores=NC, num_subcores=NS)


def _fill_rows(ref, nrows, value):
    width = ref.shape[1]

    def body(i, _):
        for j in range(width // 16):
            ref[i, pl.ds(j * 16, 16)] = jnp.full((16,), value, _F32)
        return 0

    lax.fori_loop(0, nrows, body, 0)


def _zero_acc(zrow, acc, s, width_rows):
    nz = zrow.shape[0]

    def body(i, _):
        pltpu.sync_copy(zrow, acc.at[pl.ds(s * ROWS_PER_SUB + i * nz, nz), :])
        return 0

    lax.fori_loop(0, ROWS_PER_SUB // nz, body, 0)


NSLOT = 2
NGRP = NBLK // NSLOT


def _segsum_body(two_tables, *refs):
    n_in = 5 if two_tables else 3
    tabs = refs[:n_in]
    out = refs[n_in]
    rest = list(refs[n_in + 1:])
    if two_tables:
        tab_a, idx_a, tab_b, idx_b, dst = tabs
    else:
        tab_a, idx_a, dst = tabs

    def take(n):
        nonlocal rest
        got, rest = rest[:n], rest[n:]
        return got

    ia = take(NSLOT)
    ib = take(NSLOT) if two_tables else None
    dv = take(NSLOT)
    ia16, = take(1)
    if two_tables:
        ib16, = take(1)
    dv16, = take(1)
    ra = take(NSLOT)
    rb = take(NSLOT) if two_tables else None
    ra16, = take(1)
    if two_tables:
        rb16, = take(1)
    zrow, = take(1)
    acc, = take(1)
    sidx = take(NSLOT)
    sga = take(NSLOT)
    sgb = take(NSLOT) if two_tables else None
    ssc = take(NSLOT)
    core = lax.axis_index("c")
    sid = lax.axis_index("s")
    _fill_rows(zrow, zrow.shape[0], 0.0)

    for k in range(2):
        chunk = core * 2 + k
        off = chunk * NP
        _zero_acc(zrow, acc, sid, ROWS_PER_SUB)
        plsc.subcore_barrier()

        def issue_idx(bb, t):
            base = sid * PER_SUB + bb * BLK
            pltpu.make_async_copy(idx_a.at[pl.ds(base, BLK)], ia[t],
                                  sidx[t]).start()
            if two_tables:
                pltpu.make_async_copy(idx_b.at[pl.ds(base, BLK)], ib[t],
                                      sidx[t]).start()
            pltpu.make_async_copy(dst.at[pl.ds(base, BLK)], dv[t],
                                  sidx[t]).start()

        def wait_idx_offset(bb, t):
            base = sid * PER_SUB + bb * BLK
            pltpu.make_async_copy(idx_a.at[pl.ds(base, BLK)], ia[t],
                                  sidx[t]).wait()
            if two_tables:
                pltpu.make_async_copy(idx_b.at[pl.ds(base, BLK)], ib[t],
                                      sidx[t]).wait()
            pltpu.make_async_copy(dst.at[pl.ds(base, BLK)], dv[t],
                                  sidx[t]).wait()
            for j in range(BLK // 16):
                sl = pl.ds(j * 16, 16)
                ia[t][sl] = ia[t][sl] + off
                if two_tables:
                    ib[t][sl] = ib[t][sl] + off

        def issue_gathers(t):
            pltpu.make_async_copy(tab_a.at[ia[t]], ra[t], sga[t]).start()
            if two_tables:
                pltpu.make_async_copy(tab_b.at[ib[t]], rb[t], sgb[t]).start()

        def process(t):
            pltpu.make_async_copy(tab_a.at[ia[t]], ra[t], sga[t]).wait()
            if two_tables:
                pltpu.make_async_copy(tab_b.at[ib[t]], rb[t], sgb[t]).wait()

                def addrow(i, _):
                    for j in range(8):
                        sl2 = pl.ds(j * 16, 16)
                        ra[t][i, sl2] = ra[t][i, sl2] + rb[t][i, sl2]
                    return 0

                lax.fori_loop(0, BLK, addrow, 0)
            pltpu.make_async_copy(ra[t], acc.at[dv[t]],
                                  ssc[t]).start(add=True)

        def wait_sc(t):
            pltpu.make_async_copy(ra[t], acc.at[dv[t]], ssc[t]).wait()

        for t in range(NSLOT):
            issue_idx(t, t)
            wait_idx_offset(t, t)
            issue_gathers(t)

        def grp(i, _):
            for t in range(NSLOT):
                process(t)
            nxt = (i + 1) * NSLOT
            nxt = jnp.where(nxt >= NBLK, 0, nxt)
            for t in range(NSLOT):
                wait_sc(t)
                issue_idx(nxt + t, t)
                wait_idx_offset(nxt + t, t)
                issue_gathers(t)
            return 0

        lax.fori_loop(0, NGRP, grp, 0)

        for t in range(NSLOT):
            pltpu.make_async_copy(tab_a.at[ia[t]], ra[t], sga[t]).wait()
            if two_tables:
                pltpu.make_async_copy(tab_b.at[ib[t]], rb[t], sgb[t]).wait()

        base = sid * PER_SUB + NBLK * BLK
        pltpu.sync_copy(idx_a.at[pl.ds(base, 16)], ia16)
        pltpu.sync_copy(dst.at[pl.ds(base, 16)], dv16)
        ia16[...] = ia16[...] + off
        pltpu.sync_copy(tab_a.at[ia16], ra16)
        if two_tables:
            pltpu.sync_copy(idx_b.at[pl.ds(base, 16)], ib16)
            ib16[...] = ib16[...] + off
            pltpu.sync_copy(tab_b.at[ib16], rb16)

            def addrow16(i, _):
                for j in range(8):
                    sl2 = pl.ds(j * 16, 16)
                    ra16[i, sl2] = ra16[i, sl2] + rb16[i, sl2]
                return 0

            lax.fori_loop(0, 16, addrow16, 0)
        pltpu.sync_copy(ra16, acc.at[dv16], add=True)

        plsc.subcore_barrier()
        pltpu.sync_copy(acc.at[pl.ds(sid * ROWS_PER_SUB, ROWS_PER_SUB), :],
                        out.at[chunk,
                               pl.ds(sid * ROWS_PER_SUB, ROWS_PER_SUB), :])
        plsc.subcore_barrier()


def _make_segsum(two_tables, interpret=False):
    i32v = pltpu.VMEM((BLK,), jnp.int32)
    i16v = pltpu.VMEM((16,), jnp.int32)
    rowv = pltpu.VMEM((BLK, F), _F32)
    row16 = pltpu.VMEM((16, F), _F32)
    sem = pltpu.SemaphoreType.DMA
    scratch = [i32v] * NSLOT
    if two_tables:
        scratch += [i32v] * NSLOT
    scratch += [i32v] * NSLOT
    scratch += [i16v] * (3 if two_tables else 2)
    scratch += [rowv] * NSLOT
    if two_tables:
        scratch += [rowv] * NSLOT
    scratch += [row16] * (2 if two_tables else 1)
    scratch += [pltpu.VMEM((16, F), _F32)]
    scratch += [pltpu.VMEM_SHARED((NP, F), _F32)]
    scratch += [sem] * (NSLOT * (4 if two_tables else 3))
    return pl.kernel(
        functools.partial(_segsum_body, two_tables),
        out_type=jax.ShapeDtypeStruct((NCH, NP, F), _F32),
        mesh=_mesh(),
        scratch_types=scratch,
        interpret=interpret,
    )


def _count_body(d0, d1, d2, d3, out, dv, dv16, ones, ones16, zrow, acc):
    core = lax.axis_index("c")
    s = lax.axis_index("s")
    _fill_rows(zrow, zrow.shape[0], 0.0)
    _fill_rows(ones, 128, 1.0)
    _fill_rows(ones16, 16, 1.0)
    dsts = (d0, d1, d2, d3)

    for k in range(2):
        job = core * 2 + k
        _zero_acc(zrow, acc, s, ROWS_PER_SUB)
        plsc.subcore_barrier()
        for jj in range(4):
            @pl.when(job == jj)
            def _scan(dref=dsts[jj]):
                def blk(b, _):
                    base = s * PER_SUB + b * 128
                    pltpu.sync_copy(dref.at[pl.ds(base, 128)], dv)
                    pltpu.sync_copy(ones, acc.at[dv], add=True)
                    return 0
                lax.fori_loop(0, NBLK_CNT, blk, 0)
                base = s * PER_SUB + NBLK_CNT * 128
                pltpu.sync_copy(dref.at[pl.ds(base, 16)], dv16)
                pltpu.sync_copy(ones16, acc.at[dv16], add=True)
        plsc.subcore_barrier()
        pltpu.sync_copy(acc.at[pl.ds(s * ROWS_PER_SUB, ROWS_PER_SUB), :],
                        out.at[job, pl.ds(s * ROWS_PER_SUB, ROWS_PER_SUB), :])
        plsc.subcore_barrier()


@functools.lru_cache(maxsize=None)
def _make_count(interpret=False):
    return pl.kernel(
        _count_body,
        out_type=jax.ShapeDtypeStruct((4, NP, F), _F32),
        mesh=_mesh(),
        scratch_types=[
            pltpu.VMEM((128,), jnp.int32),
            pltpu.VMEM((16,), jnp.int32),
            pltpu.VMEM((128, F), _F32),
            pltpu.VMEM((16, F), _F32),
            pltpu.VMEM((64, F), _F32),
            pltpu.VMEM_SHARED((NP, F), _F32),
        ],
        interpret=interpret,
    )


def kernel(x_win, x_edge, edge_index_near, edge_index_close, edge_index_sim,
           ij2idx_near, ij2idx_close, ij2idx_sim, edge_edge_index,
           W_pre_win, W_post_win, W_pre_edge, W_post_edge,
           W_rel_self, W_rel_nbr, pool_q, pool_W,
           edge_Wl, edge_Wr, edge_b, W_out, b_out):
    L = W_rel_self.shape[0]
    i32 = jnp.int32
    rels = [
        (edge_index_near[0].astype(i32), edge_index_near[1].astype(i32),
         ij2idx_near.astype(i32)),
        (edge_index_close[0].astype(i32), edge_index_close[1].astype(i32),
         ij2idx_close.astype(i32)),
        (edge_index_sim[0].astype(i32), edge_index_sim[1].astype(i32),
         ij2idx_sim.astype(i32)),
    ]
    es = edge_edge_index[0].astype(i32)
    ed = edge_edge_index[1].astype(i32)

    xw = _make_pre(H)(x_win, W_pre_win, W_post_win)
    xe = _make_pre(x_edge.shape[1])(x_edge, W_pre_edge, W_post_edge)

    cnt = _make_count()(rels[0][1], rels[1][1], rels[2][1], ed)

    pool = _make_pool()
    seg2 = _make_segsum(True)
    seg1 = _make_segsum(False)
    comb = _make_combine()
    eupd = _make_edge_upd()

    for l in range(L):
        gw = pool(xw, pool_q[l][None, :], pool_W[l], W_rel_nbr[l])
        xw_flat = xw.reshape(NCH * NP, F)
        xe_flat = xe.reshape(NCH * NP, F)
        s_aggr = [seg2(xw_flat, src, xe_flat, ij, dst)
                  for (src, dst, ij) in rels]
        t_aggr = seg1(xe_flat, es, ed)
        xw_new = comb(xw, s_aggr[0], s_aggr[1], s_aggr[2], cnt,
                      W_rel_self[l], W_rel_nbr[l], gw)
        xe = eupd(t_aggr, cnt, xe, edge_Wl[l], edge_Wr[l],
                  edge_b[l][None, :])
        xw = xw_new

    return _make_out()(xw, W_out, b_out[None, :])


# speedup vs baseline: 3.7620x; 1.0502x over previous
import functools

import jax
import jax.numpy as jnp
from jax import lax
from jax.experimental import pallas as pl
from jax.experimental.pallas import tpu as pltpu
from jax.experimental.pallas import tpu_sc as plsc

N = 10000
NP = 10240
E = 160000
H = 512
NCH = 4
F = 128
OUT = 250
KE_PAD = 4224

NC, NS = 2, 16
PER_SUB = E // NS
BLK = 80
NBLK_CNT = PER_SUB // 128
NBLK = PER_SUB // BLK
ROWS_PER_SUB = NP // NS

_F32 = jnp.float32


def _leaky(x):
    return jnp.where(x > 0, x, 0.2 * x)


def _pre_body(x_ref, w1_ref, w2_ref, out_ref):
    m = pl.program_id(0)
    h = _leaky(jnp.dot(x_ref[...], w1_ref[...], preferred_element_type=_F32))
    y = _leaky(jnp.dot(h, w2_ref[...], preferred_element_type=_F32))
    rows = m * 256 + lax.broadcasted_iota(jnp.int32, (256, 1), 0)
    y = jnp.where(rows < N, y, 0.0)
    for c in range(NCH):
        out_ref[c] = y[:, c * F:(c + 1) * F]


@functools.lru_cache(maxsize=None)
def _make_pre(k_dim, interpret=False):
    return pl.pallas_call(
        _pre_body,
        grid=(NP // 256,),
        in_specs=[
            pl.BlockSpec((256, k_dim), lambda m: (m, 0)),
            pl.BlockSpec((k_dim, H), lambda m: (0, 0)),
            pl.BlockSpec((H, H), lambda m: (0, 0)),
        ],
        out_specs=pl.BlockSpec((NCH, 256, F), lambda m: (0, m, 0)),
        out_shape=jax.ShapeDtypeStruct((NCH, NP, F), _F32),
        interpret=interpret,
    )


def _pool_body(xw_ref, q_ref, pw_ref, wn_ref, out_ref):
    rows = lax.broadcasted_iota(jnp.int32, (NP, 1), 0)
    mask = rows < N
    scores = jnp.zeros((NP, 1), _F32)
    for c in range(NCH):
        qc = q_ref[0, c * F:(c + 1) * F][:, None]
        scores += jnp.dot(xw_ref[c], qc, preferred_element_type=_F32)
    scores = scores * (1.0 / jnp.sqrt(jnp.float32(H)))
    sm = jnp.where(mask, scores, -1e30)
    mx = jnp.max(sm)
    e = jnp.where(mask, jnp.exp(sm - mx), 0.0)
    att = e / jnp.sum(e)
    g = jnp.zeros((1, H), _F32)
    for c in range(NCH):
        gpre = jnp.sum(att * xw_ref[c], axis=0, keepdims=True)
        g += jnp.dot(gpre, pw_ref[c * F:(c + 1) * F, :],
                     preferred_element_type=_F32)
    gws = [jnp.dot(g, wn_ref[r], preferred_element_type=_F32)
           for r in range(3)]
    out_ref[...] = jnp.concatenate(gws + [jnp.zeros((5, H), _F32)], axis=0)


@functools.lru_cache(maxsize=None)
def _make_pool(interpret=False):
    return pl.pallas_call(
        _pool_body,
        in_specs=[
            pl.BlockSpec((NCH, NP, F), lambda: (0, 0, 0)),
            pl.BlockSpec((1, H), lambda: (0, 0)),
            pl.BlockSpec((H, H), lambda: (0, 0)),
            pl.BlockSpec((3, H, H), lambda: (0, 0, 0)),
        ],
        out_specs=pl.BlockSpec((8, H), lambda: (0, 0)),
        out_shape=jax.ShapeDtypeStruct((8, H), _F32),
        interpret=interpret,
    )


def _combine_body(xw_ref, s0_ref, s1_ref, s2_ref, cnt_ref, wself_ref,
                  wnbr_ref, gw_ref, out_ref):
    m = pl.program_id(0)
    ws = wself_ref[0] + wself_ref[1] + wself_ref[2]
    acc = jnp.zeros((256, F), _F32)
    for c in range(NCH):
        acc += jnp.dot(xw_ref[c], ws[c * F:(c + 1) * F, :],
                       preferred_element_type=_F32)
    s_refs = (s0_ref, s1_ref, s2_ref)
    for r in range(3):
        cr = cnt_ref[r, :, 0:1]
        invc = 1.0 / jnp.maximum(cr, 1.0)
        dr = jnp.where(cr > 0, 1.0, 0.0)
        for c in range(NCH):
            acc += jnp.dot(s_refs[r][c] * invc,
                           wnbr_ref[r, c * F:(c + 1) * F, :],
                           preferred_element_type=_F32)
        acc += dr * gw_ref[r, :][None, :]
    y = _leaky(acc * (1.0 / 3.0))
    rows = m * 256 + lax.broadcasted_iota(jnp.int32, (256, 1), 0)
    out_ref[0] = jnp.where(rows < N, y, 0.0)


@functools.lru_cache(maxsize=None)
def _make_combine(interpret=False):
    sblock = pl.BlockSpec((NCH, 256, F), lambda m, n: (0, m, 0))
    return pl.pallas_call(
        _combine_body,
        grid=(NP // 256, NCH),
        in_specs=[
            sblock, sblock, sblock, sblock,
            pl.BlockSpec((NCH, 256, F), lambda m, n: (0, m, 0)),
            pl.BlockSpec((3, H, F), lambda m, n: (0, 0, n)),
            pl.BlockSpec((3, H, F), lambda m, n: (0, 0, n)),
            pl.BlockSpec((8, F), lambda m, n: (0, n)),
        ],
        out_specs=pl.BlockSpec((1, 256, F), lambda m, n: (n, m, 0)),
        out_shape=jax.ShapeDtypeStruct((NCH, NP, F), _F32),
        interpret=interpret,
    )


def _edge_upd_body(t_ref, cnt_ref, xe_ref, wl_ref, wr_ref, b_ref, out_ref):
    m = pl.program_id(0)
    cr = cnt_ref[3, :, 0:1]
    invc = 1.0 / jnp.maximum(cr, 1.0)
    acc = jnp.zeros((256, F), _F32)
    for c in range(NCH):
        acc += jnp.dot(t_ref[c] * invc, wl_ref[c * F:(c + 1) * F, :],
                       preferred_element_type=_F32)
        acc += jnp.dot(xe_ref[c], wr_ref[c * F:(c + 1) * F, :],
                       preferred_element_type=_F32)
    y = _leaky(acc + b_ref[...])
    rows = m * 256 + lax.broadcasted_iota(jnp.int32, (256, 1), 0)
    out_ref[0] = jnp.where(rows < N, y, 0.0)


@functools.lru_cache(maxsize=None)
def _make_edge_upd(interpret=False):
    cblock = pl.BlockSpec((NCH, 256, F), lambda m, n: (0, m, 0))
    return pl.pallas_call(
        _edge_upd_body,
        grid=(NP // 256, NCH),
        in_specs=[
            cblock,
            pl.BlockSpec((NCH, 256, F), lambda m, n: (0, m, 0)),
            cblock,
            pl.BlockSpec((H, F), lambda m, n: (0, n)),
            pl.BlockSpec((H, F), lambda m, n: (0, n)),
            pl.BlockSpec((1, F), lambda m, n: (0, n)),
        ],
        out_specs=pl.BlockSpec((1, 256, F), lambda m, n: (n, m, 0)),
        out_shape=jax.ShapeDtypeStruct((NCH, NP, F), _F32),
        interpret=interpret,
    )


def _out_body(x_ref, w_ref, b_ref, out_ref):
    acc = jnp.zeros((256, OUT), _F32)
    for c in range(NCH):
        acc += jnp.dot(x_ref[c], w_ref[c * F:(c + 1) * F, :],
                       preferred_element_type=_F32)
    out_ref[...] = acc + b_ref[...]


@functools.lru_cache(maxsize=None)
def _make_out(interpret=False):
    return pl.pallas_call(
        _out_body,
        grid=(NP // 256,),
        in_specs=[
            pl.BlockSpec((NCH, 256, F), lambda m: (0, m, 0)),
            pl.BlockSpec((H, OUT), lambda m: (0, 0)),
            pl.BlockSpec((1, OUT), lambda m: (0, 0)),
        ],
        out_specs=pl.BlockSpec((256, OUT), lambda m: (m, 0)),
        out_shape=jax.ShapeDtypeStruct((N, OUT), _F32),
        interpret=interpret,
    )


@functools.lru_cache(maxsize=None)
def _mesh():
    return plsc.VectorSubcoreMesh(core_axis_name="c", subcore_axis_name="s",
                                  num_cores=NC, num_subcores=NS)


def _fill_rows(ref, nrows, value):
    width = ref.shape[1]

    def body(i, _):
        for j in range(width // 16):
            ref[i, pl.ds(j * 16, 16)] = jnp.full((16,), value, _F32)
        return 0

    lax.fori_loop(0, nrows, body, 0)


def _zero_acc(zrow, acc, s, width_rows):
    nz = zrow.shape[0]

    def body(i, _):
        pltpu.sync_copy(zrow, acc.at[pl.ds(s * ROWS_PER_SUB + i * nz, nz), :])
        return 0

    lax.fori_loop(0, ROWS_PER_SUB // nz, body, 0)


NSLOT = 2
NGRP = NBLK // NSLOT
NLEFT = NBLK - NGRP * NSLOT


def _segsum_body(two_tables, *refs):
    n_in = 5 if two_tables else 3
    tabs = refs[:n_in]
    out = refs[n_in]
    rest = list(refs[n_in + 1:])
    if two_tables:
        tab_a, idx_a, tab_b, idx_b, dst = tabs
    else:
        tab_a, idx_a, dst = tabs

    def take(n):
        nonlocal rest
        got, rest = rest[:n], rest[n:]
        return got

    ia = take(NSLOT)
    ib = take(NSLOT) if two_tables else None
    dv = take(NSLOT)
    ra = take(NSLOT)
    rb = take(NSLOT) if two_tables else None
    zrow, = take(1)
    acc, = take(1)
    sidx = take(NSLOT)
    sga = take(NSLOT)
    sgb = take(NSLOT) if two_tables else None
    ssc = take(NSLOT)
    core = lax.axis_index("c")
    sid = lax.axis_index("s")
    _fill_rows(zrow, zrow.shape[0], 0.0)

    for k in range(2):
        chunk = core * 2 + k
        off = chunk * NP
        _zero_acc(zrow, acc, sid, ROWS_PER_SUB)
        plsc.subcore_barrier()

        def issue_idx(bb, t):
            base = sid * PER_SUB + bb * BLK
            pltpu.make_async_copy(idx_a.at[pl.ds(base, BLK)], ia[t],
                                  sidx[t]).start()
            if two_tables:
                pltpu.make_async_copy(idx_b.at[pl.ds(base, BLK)], ib[t],
                                      sidx[t]).start()
            pltpu.make_async_copy(dst.at[pl.ds(base, BLK)], dv[t],
                                  sidx[t]).start()

        def wait_idx_offset(bb, t):
            base = sid * PER_SUB + bb * BLK
            pltpu.make_async_copy(idx_a.at[pl.ds(base, BLK)], ia[t],
                                  sidx[t]).wait()
            if two_tables:
                pltpu.make_async_copy(idx_b.at[pl.ds(base, BLK)], ib[t],
                                      sidx[t]).wait()
            pltpu.make_async_copy(dst.at[pl.ds(base, BLK)], dv[t],
                                  sidx[t]).wait()
            for j in range(BLK // 16):
                sl = pl.ds(j * 16, 16)
                ia[t][sl] = ia[t][sl] + off
                if two_tables:
                    ib[t][sl] = ib[t][sl] + off

        def issue_gathers(t):
            pltpu.make_async_copy(tab_a.at[ia[t]], ra[t], sga[t]).start()
            if two_tables:
                pltpu.make_async_copy(tab_b.at[ib[t]], rb[t], sgb[t]).start()

        def process(t):
            pltpu.make_async_copy(tab_a.at[ia[t]], ra[t], sga[t]).wait()
            if two_tables:
                pltpu.make_async_copy(tab_b.at[ib[t]], rb[t], sgb[t]).wait()

                def addrow(i, _):
                    for j in range(8):
                        sl2 = pl.ds(j * 16, 16)
                        ra[t][i, sl2] = ra[t][i, sl2] + rb[t][i, sl2]
                    return 0

                lax.fori_loop(0, BLK, addrow, 0)
            pltpu.make_async_copy(ra[t], acc.at[dv[t]],
                                  ssc[t]).start(add=True)

        def wait_sc(t):
            pltpu.make_async_copy(ra[t], acc.at[dv[t]], ssc[t]).wait()

        for t in range(NSLOT):
            issue_idx(t, t)
            wait_idx_offset(t, t)
            issue_gathers(t)

        def grp(i, _):
            for t in range(NSLOT):
                process(t)
            nxt = (i + 1) * NSLOT
            for t in range(NSLOT):
                wait_sc(t)
                bbt = jnp.where(nxt + t >= NBLK, t, nxt + t)
                issue_idx(bbt, t)
                wait_idx_offset(bbt, t)
                issue_gathers(t)
            return 0

        lax.fori_loop(0, NGRP, grp, 0)

        for t in range(NLEFT):
            process(t)
            wait_sc(t)
        for t in range(NLEFT, NSLOT):
            pltpu.make_async_copy(tab_a.at[ia[t]], ra[t], sga[t]).wait()
            if two_tables:
                pltpu.make_async_copy(tab_b.at[ib[t]], rb[t], sgb[t]).wait()

        plsc.subcore_barrier()
        pltpu.sync_copy(acc.at[pl.ds(sid * ROWS_PER_SUB, ROWS_PER_SUB), :],
                        out.at[chunk,
                               pl.ds(sid * ROWS_PER_SUB, ROWS_PER_SUB), :])
        plsc.subcore_barrier()


def _make_segsum(two_tables, interpret=False):
    i32v = pltpu.VMEM((BLK,), jnp.int32)
    rowv = pltpu.VMEM((BLK, F), _F32)
    sem = pltpu.SemaphoreType.DMA
    scratch = [i32v] * NSLOT
    if two_tables:
        scratch += [i32v] * NSLOT
    scratch += [i32v] * NSLOT
    scratch += [rowv] * NSLOT
    if two_tables:
        scratch += [rowv] * NSLOT
    scratch += [pltpu.VMEM((16, F), _F32)]
    scratch += [pltpu.VMEM_SHARED((NP, F), _F32)]
    scratch += [sem] * (NSLOT * (4 if two_tables else 3))
    return pl.kernel(
        functools.partial(_segsum_body, two_tables),
        out_type=jax.ShapeDtypeStruct((NCH, NP, F), _F32),
        mesh=_mesh(),
        scratch_types=scratch,
        interpret=interpret,
    )


def _count_body(d0, d1, d2, d3, out, dv, dv16, ones, ones16, zrow, acc):
    core = lax.axis_index("c")
    s = lax.axis_index("s")
    _fill_rows(zrow, zrow.shape[0], 0.0)
    _fill_rows(ones, 128, 1.0)
    _fill_rows(ones16, 16, 1.0)
    dsts = (d0, d1, d2, d3)

    for k in range(2):
        job = core * 2 + k
        _zero_acc(zrow, acc, s, ROWS_PER_SUB)
        plsc.subcore_barrier()
        for jj in range(4):
            @pl.when(job == jj)
            def _scan(dref=dsts[jj]):
                def blk(b, _):
                    base = s * PER_SUB + b * 128
                    pltpu.sync_copy(dref.at[pl.ds(base, 128)], dv)
                    pltpu.sync_copy(ones, acc.at[dv], add=True)
                    return 0
                lax.fori_loop(0, NBLK_CNT, blk, 0)
                base = s * PER_SUB + NBLK_CNT * 128
                pltpu.sync_copy(dref.at[pl.ds(base, 16)], dv16)
                pltpu.sync_copy(ones16, acc.at[dv16], add=True)
        plsc.subcore_barrier()
        pltpu.sync_copy(acc.at[pl.ds(s * ROWS_PER_SUB, ROWS_PER_SUB), :],
                        out.at[job, pl.ds(s * ROWS_PER_SUB, ROWS_PER_SUB), :])
        plsc.subcore_barrier()


@functools.lru_cache(maxsize=None)
def _make_count(interpret=False):
    return pl.kernel(
        _count_body,
        out_type=jax.ShapeDtypeStruct((4, NP, F), _F32),
        mesh=_mesh(),
        scratch_types=[
            pltpu.VMEM((128,), jnp.int32),
            pltpu.VMEM((16,), jnp.int32),
            pltpu.VMEM((128, F), _F32),
            pltpu.VMEM((16, F), _F32),
            pltpu.VMEM((64, F), _F32),
            pltpu.VMEM_SHARED((NP, F), _F32),
        ],
        interpret=interpret,
    )


def kernel(x_win, x_edge, edge_index_near, edge_index_close, edge_index_sim,
           ij2idx_near, ij2idx_close, ij2idx_sim, edge_edge_index,
           W_pre_win, W_post_win, W_pre_edge, W_post_edge,
           W_rel_self, W_rel_nbr, pool_q, pool_W,
           edge_Wl, edge_Wr, edge_b, W_out, b_out):
    L = W_rel_self.shape[0]
    i32 = jnp.int32
    rels = [
        (edge_index_near[0].astype(i32), edge_index_near[1].astype(i32),
         ij2idx_near.astype(i32)),
        (edge_index_close[0].astype(i32), edge_index_close[1].astype(i32),
         ij2idx_close.astype(i32)),
        (edge_index_sim[0].astype(i32), edge_index_sim[1].astype(i32),
         ij2idx_sim.astype(i32)),
    ]
    es = edge_edge_index[0].astype(i32)
    ed = edge_edge_index[1].astype(i32)

    xw = _make_pre(H)(x_win, W_pre_win, W_post_win)
    xe = _make_pre(x_edge.shape[1])(x_edge, W_pre_edge, W_post_edge)

    cnt = _make_count()(rels[0][1], rels[1][1], rels[2][1], ed)

    pool = _make_pool()
    seg2 = _make_segsum(True)
    seg1 = _make_segsum(False)
    comb = _make_combine()
    eupd = _make_edge_upd()

    for l in range(L):
        gw = pool(xw, pool_q[l][None, :], pool_W[l], W_rel_nbr[l])
        xw_flat = xw.reshape(NCH * NP, F)
        xe_flat = xe.reshape(NCH * NP, F)
        s_aggr = [seg2(xw_flat, src, xe_flat, ij, dst)
                  for (src, dst, ij) in rels]
        t_aggr = seg1(xe_flat, es, ed)
        xw_new = comb(xw, s_aggr[0], s_aggr[1], s_aggr[2], cnt,
                      W_rel_self[l], W_rel_nbr[l], gw)
        xe = eupd(t_aggr, cnt, xe, edge_Wl[l], edge_Wr[l],
                  edge_b[l][None, :])
        xw = xw_new

    return _make_out()(xw, W_out, b_out[None, :])
